# Initial kernel scaffold; baseline (speedup 1.0000x reference)
import jax
import jax.numpy as jnp
from jax.experimental import pallas as pl


def kernel(x, edge_index, batch, W1, b1, W2, b2, eps, gamma, beta):
    raise NotImplementedError("write your pallas kernel here")


# speedup vs baseline: 13.2964x; 13.2964x over previous
import functools

import jax
import jax.numpy as jnp
from jax import lax
from jax.experimental import pallas as pl
from jax.experimental.pallas import tpu as pltpu
from jax.experimental.pallas import tpu_sc as plsc

_N_NODES = 10000
_N_EDGES = 320000
_D = 128
_N_GRAPHS = 64

_EW = 128
_TPS = 16
_NW = 32
_RPW = 80
_CHUNK = 40
_ROWS = _NW * _RPW
_N_PAD = _ROWS * _EW - _N_EDGES
_N_DUMMY = 16
_NPT = 624
_NPT_TAIL = _N_NODES - _NPT * _TPS


def _sc_agg(x, src2d, dst2d, zeros):
    mesh = plsc.VectorSubcoreMesh(core_axis_name="c", subcore_axis_name="s")

    @functools.partial(
        pl.kernel,
        out_type=jax.ShapeDtypeStruct((2 * _N_NODES, _D), jnp.float32),
        mesh=mesh,
        scratch_types=[
            pltpu.VMEM((_CHUNK, _EW), jnp.int32),
            pltpu.VMEM((_CHUNK, _EW), jnp.int32),
            pltpu.VMEM((_EW, _D), jnp.float32),
            pltpu.VMEM((_EW, _D), jnp.float32),
            pltpu.VMEM_SHARED((_N_NODES + _N_DUMMY, _D), jnp.float32),
            pltpu.SemaphoreType.DMA,
            pltpu.SemaphoreType.DMA,
        ],
    )
    def k(x_hbm, src_hbm, dst_hbm, zero_hbm, out_hbm,
          src_v, dst_v, buf0, buf1, agg, sem0, sem1):
        c = lax.axis_index("c")
        s = lax.axis_index("s")
        wid = c * _TPS + s

        pltpu.sync_copy(zero_hbm, agg.at[pl.ds(s * _NPT, _NPT)])

        @pl.when(s == 0)
        def _():
            pltpu.sync_copy(zero_hbm.at[pl.ds(0, _NPT_TAIL)],
                            agg.at[pl.ds(_TPS * _NPT, _NPT_TAIL)])

        plsc.subcore_barrier()

        def gather(j, buf, sem):
            pltpu.async_copy(x_hbm.at[src_v.at[j]], buf, sem)

        def gwait(j, buf, sem):
            pltpu.make_async_copy(x_hbm.at[src_v.at[j]], buf, sem).wait()

        for chunk in range(_RPW // _CHUNK):
            base = wid * _RPW + chunk * _CHUNK
            pltpu.sync_copy(src_hbm.at[pl.ds(base, _CHUNK)], src_v)
            pltpu.sync_copy(dst_hbm.at[pl.ds(base, _CHUNK)], dst_v)

            gather(0, buf0, sem0)

            def body(i, carry):
                j0 = 2 * i
                gather(j0 + 1, buf1, sem1)
                gwait(j0, buf0, sem0)
                pltpu.sync_copy(buf0, agg.at[dst_v.at[j0]], add=True)

                @pl.when(j0 + 2 < _CHUNK)
                def _():
                    gather(j0 + 2, buf0, sem0)

                gwait(j0 + 1, buf1, sem1)
                pltpu.sync_copy(buf1, agg.at[dst_v.at[j0 + 1]], add=True)
                return carry

            lax.fori_loop(0, _CHUNK // 2, body, 0)

        plsc.subcore_barrier()

        off = c * _N_NODES + s * _NPT
        pltpu.sync_copy(agg.at[pl.ds(s * _NPT, _NPT)], out_hbm.at[pl.ds(off, _NPT)])

        @pl.when(s == 0)
        def _():
            pltpu.sync_copy(agg.at[pl.ds(_TPS * _NPT, _NPT_TAIL)],
                            out_hbm.at[pl.ds(c * _N_NODES + _TPS * _NPT, _NPT_TAIL)])

    return k(x, src2d, dst2d, zeros)


def _tc_head(x, aggs, W1, b1, W2, b2, eps, gamma, beta, ng, nb, batch_row):
    def body(x_ref, aggs_ref, w1_ref, b1_ref, w2_ref, b2_ref, eps_ref,
             g_ref, be_ref, ng_ref, nb_ref, batch_ref, o_ref):
        agg = aggs_ref[0:_N_NODES, :] + aggs_ref[_N_NODES:2 * _N_NODES, :]
        h = (1.0 + eps_ref[...]) * x_ref[...] + agg
        h = jnp.dot(h, w1_ref[...], preferred_element_type=jnp.float32) + b1_ref[...]
        h = jnp.maximum(h, 0.0)
        h = jnp.dot(h, w2_ref[...], preferred_element_type=jnp.float32) + b2_ref[...]
        sg = jnp.log1p(jnp.exp(g_ref[...]))
        sb = jnp.log1p(jnp.exp(be_ref[...]))
        h = (1.0 + sg * ng_ref[...]) * h + sb * nb_ref[...]
        h = h * jax.nn.sigmoid(h)
        oh = (batch_ref[...] == lax.broadcasted_iota(
            jnp.int32, (_N_GRAPHS, _N_NODES), 0)).astype(jnp.float32)
        sums = jnp.dot(oh, h, preferred_element_type=jnp.float32)
        counts = jnp.sum(oh, axis=1, keepdims=True)
        o_ref[...] = sums / jnp.maximum(counts, 1.0)

    return pl.pallas_call(
        body,
        out_shape=jax.ShapeDtypeStruct((_N_GRAPHS, _D), jnp.float32),
    )(x, aggs, W1, b1, W2, b2, eps, gamma, beta, ng, nb, batch_row)


def kernel(x, edge_index, batch, W1, b1, W2, b2, eps, gamma, beta):
    e32 = edge_index.astype(jnp.int32)
    pad_src = jnp.arange(_N_PAD, dtype=jnp.int32) % _N_NODES
    pad_dst = _N_NODES + jnp.arange(_N_PAD, dtype=jnp.int32) % _N_DUMMY
    src2d = jnp.concatenate([e32[0], pad_src]).reshape(_ROWS, _EW)
    dst2d = jnp.concatenate([e32[1], pad_dst]).reshape(_ROWS, _EW)
    zeros = jnp.zeros((_NPT, _D), dtype=jnp.float32)

    aggs = _sc_agg(x, src2d, dst2d, zeros)

    nk = jax.random.key(42)
    ng = jax.random.normal(jax.random.fold_in(nk, 1), (1, _D), dtype=jnp.float32)
    nb = jax.random.normal(jax.random.fold_in(nk, 2), (1, _D), dtype=jnp.float32)

    return _tc_head(x, aggs, W1, b1.reshape(1, _D), W2, b2.reshape(1, _D),
                    eps.reshape(1, 1), gamma.reshape(1, _D), beta.reshape(1, _D),
                    ng, nb, batch.astype(jnp.int32).reshape(1, _N_NODES))
